# baseline (speedup 1.0000x reference)
import jax
import jax.numpy as jnp
from jax.experimental import pallas as pl
from jax.experimental.pallas import tpu as pltpu
from jax.experimental.pallas import tpu_sc as plsc

_GATHER_WINDOW = 128
_BV = 4096


def _sc_gather_packed(packed_table, packed_idx):
    n = packed_idx.shape[0]
    idx2d = packed_idx.reshape(1, n)
    mesh = plsc.VectorSubcoreMesh(core_axis_name="core", subcore_axis_name="subcore")

    @pl.kernel(
        out_type=jax.ShapeDtypeStruct((n, packed_table.shape[1]), packed_table.dtype),
        mesh=mesh,
    )
    def gather_kernel(table_hbm, idx_hbm, out_hbm):
        def body(idx_vmem, out_vmem):
            pltpu.sync_copy(table_hbm.at[idx_vmem.at[0]], out_vmem)

        pltpu.emit_pipeline(
            body,
            grid=(n // _GATHER_WINDOW,),
            in_specs=[pl.BlockSpec((1, _GATHER_WINDOW), index_map=lambda i: (0, i))],
            out_specs=[
                pl.BlockSpec(
                    (_GATHER_WINDOW, packed_table.shape[1]),
                    index_map=lambda i: (i, 0),
                )
            ],
            core_axis_name="subcore",
            dimension_semantics=(pltpu.PARALLEL,),
        )(idx_hbm, out_hbm)

    return gather_kernel(packed_table, idx2d)


def _tc_project_t(h_packed, parity, W, bcol):
    batch = h_packed.shape[0]
    dim, vocab = W.shape
    grid = pl.cdiv(vocab, _BV)

    def body(hp_ref, par_ref, w_ref, b_ref, o_ref):
        h = jnp.where(par_ref[...] != 0, hp_ref[:, dim:], hp_ref[:, :dim])
        w_aug = jnp.concatenate([w_ref[...], b_ref[...]], axis=0)
        h_aug = jnp.concatenate(
            [h, jnp.ones((batch, 1), jnp.float32)], axis=1
        )
        o_ref[...] = jax.lax.dot_general(
            w_aug, h_aug,
            (((0,), (1,)), ((), ())),
            preferred_element_type=jnp.float32,
        )

    return pl.pallas_call(
        body,
        grid=(grid,),
        in_specs=[
            pl.BlockSpec((batch, 2 * dim), lambda k: (0, 0)),
            pl.BlockSpec((batch, 1), lambda k: (0, 0)),
            pl.BlockSpec((dim, _BV), lambda k: (0, k)),
            pl.BlockSpec((1, _BV), lambda k: (0, k)),
        ],
        out_specs=pl.BlockSpec((_BV, batch), lambda k: (k, 0)),
        out_shape=jax.ShapeDtypeStruct((vocab, batch), jnp.float32),
        compiler_params=pltpu.CompilerParams(
            dimension_semantics=("arbitrary",),
        ),
    )(h_packed, parity, W, bcol)


_PACK_ROWS = 400


def _tc_pack_table(embed_table):
    vocab_rows, dim = embed_table.shape
    half_blocks = (vocab_rows // 2) // _PACK_ROWS

    def body(lo_ref, hi_ref, o_ref):
        o_ref[...] = jnp.concatenate([lo_ref[...], hi_ref[...]], axis=1)

    return pl.pallas_call(
        body,
        grid=(half_blocks,),
        in_specs=[
            pl.BlockSpec((_PACK_ROWS, dim), lambda g: (g, 0)),
            pl.BlockSpec((_PACK_ROWS, dim), lambda g: (g + half_blocks, 0)),
        ],
        out_specs=pl.BlockSpec((_PACK_ROWS, 2 * dim), lambda g: (g, 0)),
        out_shape=jax.ShapeDtypeStruct((vocab_rows // 2, 2 * dim), embed_table.dtype),
        compiler_params=pltpu.CompilerParams(
            dimension_semantics=("arbitrary",),
        ),
    )(embed_table, embed_table)


def kernel(input_ids, embed_table, W, b):
    vocab_rows, dim = embed_table.shape
    half = vocab_rows // 2
    packed_table = _tc_pack_table(embed_table)
    in_hi = (input_ids >= half).astype(jnp.int32)
    h_packed = _sc_gather_packed(packed_table, input_ids - half * in_hi)
    logits_t = _tc_project_t(h_packed, in_hi.reshape(-1, 1), W, b.reshape(1, -1))
    return jnp.transpose(logits_t)


# speedup vs baseline: 1.2664x; 1.2664x over previous
import jax
import jax.numpy as jnp
from jax.experimental import pallas as pl
from jax.experimental.pallas import tpu as pltpu
from jax.experimental.pallas import tpu_sc as plsc

_GATHER_WINDOW = 128
_BV = 4096


def _sc_gather_packed(packed_table, packed_idx):
    n = packed_idx.shape[0]
    idx2d = packed_idx.reshape(1, n)
    mesh = plsc.VectorSubcoreMesh(core_axis_name="core", subcore_axis_name="subcore")

    @pl.kernel(
        out_type=jax.ShapeDtypeStruct((n, packed_table.shape[1]), packed_table.dtype),
        mesh=mesh,
    )
    def gather_kernel(table_hbm, idx_hbm, out_hbm):
        def body(idx_vmem, out_vmem):
            pltpu.sync_copy(table_hbm.at[idx_vmem.at[0]], out_vmem)

        pltpu.emit_pipeline(
            body,
            grid=(n // _GATHER_WINDOW,),
            in_specs=[pl.BlockSpec((1, _GATHER_WINDOW), index_map=lambda i: (0, i))],
            out_specs=[
                pl.BlockSpec(
                    (_GATHER_WINDOW, packed_table.shape[1]),
                    index_map=lambda i: (i, 0),
                )
            ],
            core_axis_name="subcore",
            dimension_semantics=(pltpu.PARALLEL,),
        )(idx_hbm, out_hbm)

    return gather_kernel(packed_table, idx2d)


def _tc_project_t(h_packed, parity, W, bcol):
    batch = h_packed.shape[0]
    dim, vocab = W.shape
    grid = pl.cdiv(vocab, _BV)

    def body(hp_ref, par_ref, w_ref, b_ref, o_ref):
        h = jnp.where(par_ref[...] != 0, hp_ref[:, dim:], hp_ref[:, :dim])
        w_aug = jnp.concatenate([w_ref[...], b_ref[...]], axis=0)
        h_aug = jnp.concatenate(
            [h, jnp.ones((batch, 1), jnp.float32)], axis=1
        )
        o_ref[...] = jax.lax.dot_general(
            w_aug, h_aug,
            (((0,), (1,)), ((), ())),
            preferred_element_type=jnp.float32,
        )

    return pl.pallas_call(
        body,
        grid=(grid,),
        in_specs=[
            pl.BlockSpec((batch, 2 * dim), lambda k: (0, 0)),
            pl.BlockSpec((batch, 1), lambda k: (0, 0)),
            pl.BlockSpec((dim, _BV), lambda k: (0, k)),
            pl.BlockSpec((1, _BV), lambda k: (0, k)),
        ],
        out_specs=pl.BlockSpec((_BV, batch), lambda k: (k, 0)),
        out_shape=jax.ShapeDtypeStruct((vocab, batch), jnp.float32),
        compiler_params=pltpu.CompilerParams(
            dimension_semantics=("parallel",),
        ),
    )(h_packed, parity, W, bcol)


def kernel(input_ids, embed_table, W, b):
    vocab_rows, dim = embed_table.shape
    packed_table = embed_table.reshape(vocab_rows // 2, 2 * dim)
    h_packed = _sc_gather_packed(packed_table, input_ids // 2)
    parity = (input_ids % 2).astype(jnp.int32).reshape(-1, 1)
    logits_t = _tc_project_t(h_packed, parity, W, b.reshape(1, -1))
    return jnp.transpose(logits_t)
